# baseline (speedup 1.0000x reference)
import jax
import jax.numpy as jnp
from jax.experimental import pallas as pl
from jax.experimental.pallas import tpu as pltpu


def _tc_body(vis_ref, tac_ref, wv_ref, wt_ref, bp_ref, src_ref, dst_ref, ei_ref,
             x_ref, attr_ref, eib_ref, m_ref):
    b = pl.program_id(0)
    E, N = m_ref.shape

    @pl.when(b == 0)
    def _build_incidence():
        ids = jax.lax.broadcasted_iota(jnp.int32, (E, N), 1)
        s = src_ref[:, 0:1]
        d = dst_ref[:, 0:1]
        m_ref[...] = (ids == s).astype(jnp.float32) - (ids == d).astype(jnp.float32)

    cv = vis_ref.shape[1]
    vT = vis_ref[0].T
    tT = tac_ref[0].T
    x_ref[0, :, 0:cv] = vT
    x_ref[0, :, cv:] = tT
    coords = (jnp.dot(vT, wv_ref[...], preferred_element_type=jnp.float32)
              + jnp.dot(tT, wt_ref[...], preferred_element_type=jnp.float32)
              + bp_ref[...])
    diff = jnp.dot(m_ref[...], coords, preferred_element_type=jnp.float32)
    dx = diff[:, 0:1]
    dy = diff[:, 1:2]
    dist = jnp.sqrt(dx * dx + dy * dy)
    w = 1.0 / (dist + 1e-6)
    attr_ref[0] = 1.0 / (1.0 + jnp.exp(-w))
    eib_ref[0] = ei_ref[...] + (b * N).astype(ei_ref.dtype)


def kernel(visual_feat, tactile_feat, Wp, bp, edge_index):
    B, Cv, H, W = visual_feat.shape
    Ct = tactile_feat.shape[1]
    C = Cv + Ct
    N = H * W
    E = edge_index.shape[1]

    vis = visual_feat.reshape(B, Cv, N)
    tac = tactile_feat.reshape(B, Ct, N)
    wv = Wp[:Cv]
    wt = Wp[Cv:]
    bp2 = bp.reshape(1, 2)
    src = edge_index[0].reshape(E, 1).astype(jnp.int32)
    dst = edge_index[1].reshape(E, 1).astype(jnp.int32)

    in_specs = [
            pl.BlockSpec((1, Cv, N), lambda b: (b, 0, 0)),
            pl.BlockSpec((1, Ct, N), lambda b: (b, 0, 0)),
            pl.BlockSpec((Cv, 2), lambda b: (0, 0)),
            pl.BlockSpec((Ct, 2), lambda b: (0, 0)),
            pl.BlockSpec((1, 2), lambda b: (0, 0)),
            pl.BlockSpec((E, 1), lambda b: (0, 0)),
            pl.BlockSpec((E, 1), lambda b: (0, 0)),
            pl.BlockSpec((2, E), lambda b: (0, 0)),
    ]
    out_specs = [
            pl.BlockSpec((1, N, C), lambda b: (b, 0, 0)),
            pl.BlockSpec((1, E, 1), lambda b: (b, 0, 0)),
            pl.BlockSpec((1, 2, E), lambda b: (b, 0, 0)),
    ]

    x_out, attr_out, eib_out = pl.pallas_call(
        _tc_body,
        grid=(B,),
        in_specs=in_specs,
        out_specs=out_specs,
        out_shape=[
            jax.ShapeDtypeStruct((B, N, C), jnp.float32),
            jax.ShapeDtypeStruct((B, E, 1), jnp.float32),
            jax.ShapeDtypeStruct((B, 2, E), edge_index.dtype),
        ],
        scratch_shapes=[pltpu.VMEM((E, N), jnp.float32)],
    )(vis, tac, wv, wt, bp2, src, dst, edge_index)

    x_batched = x_out.reshape(B * N, C)
    edge_index_batched = eib_out.transpose(1, 0, 2).reshape(2, B * E)
    edge_attr_batched = attr_out.reshape(B * E, 1)
    return (x_batched, edge_index_batched, edge_attr_batched)
